# Initial kernel scaffold; baseline (speedup 1.0000x reference)
import jax
import jax.numpy as jnp
from jax.experimental import pallas as pl


def kernel(pcs1, pcs2):
    raise NotImplementedError("write your pallas kernel here")


# speedup vs baseline: 1.1974x; 1.1974x over previous
import functools

import jax
import jax.numpy as jnp
from jax.experimental import pallas as pl
from jax.experimental.pallas import tpu as pltpu

B = 4
N = 4096
M = 4096
TN = 1024


def _chamfer_body(p1_ref, p2_ref, out_ref, d2_ref):
    b = pl.program_id(0)
    i = pl.program_id(1)
    ni = pl.num_programs(1)

    a = p1_ref[0]
    k = p2_ref[0]

    sq1 = jnp.sum(a * a, axis=0)
    sq2 = jnp.sum(k * k, axis=0)
    inner = jax.lax.dot_general(
        a, k, (((0,), (0,)), ((), ())),
        preferred_element_type=jnp.float32,
        precision=jax.lax.Precision.DEFAULT,
    )
    d = sq1[:, None] + sq2[None, :] - 2.0 * inner
    d = jnp.maximum(d, 0.0)

    @pl.when(jnp.logical_and(b == 0, i == 0))
    def _():
        out_ref[0, 0] = 0.0

    d1 = jnp.min(d, axis=1)
    out_ref[0, 0] += jnp.sum(jnp.sqrt(d1)) * (0.5 / (B * N))

    colmin = jnp.min(d, axis=0)

    @pl.when(i == 0)
    def _():
        d2_ref[0, :] = colmin

    @pl.when(i > 0)
    def _():
        d2_ref[0, :] = jnp.minimum(d2_ref[0, :], colmin)

    @pl.when(i == ni - 1)
    def _():
        out_ref[0, 0] += jnp.sum(jnp.sqrt(d2_ref[0, :])) * (0.5 / (B * M))


@jax.jit
def kernel(pcs1, pcs2):
    p1t = jnp.transpose(pcs1, (0, 2, 1))
    p2t = jnp.transpose(pcs2, (0, 2, 1))

    out = pl.pallas_call(
        _chamfer_body,
        grid=(B, N // TN),
        in_specs=[
            pl.BlockSpec((1, 3, TN), lambda b, i: (b, 0, i)),
            pl.BlockSpec((1, 3, M), lambda b, i: (b, 0, 0)),
        ],
        out_specs=pl.BlockSpec(
            (1, 1), lambda b, i: (0, 0), memory_space=pltpu.SMEM
        ),
        out_shape=jax.ShapeDtypeStruct((1, 1), jnp.float32),
        scratch_shapes=[pltpu.VMEM((1, M), jnp.float32)],
    )(p1t, p2t)
    return out[0, 0]


# speedup vs baseline: 1.4710x; 1.2284x over previous
import functools

import jax
import jax.numpy as jnp
from jax.experimental import pallas as pl
from jax.experimental.pallas import tpu as pltpu

B = 4
N = 4096
M = 4096
TN = 1024


def _chamfer_body(p1_ref, p1s_ref, p2_ref, out_ref, d2_ref):
    b = pl.program_id(0)
    i = pl.program_id(1)
    ni = pl.num_programs(1)

    am2 = p1_ref[0]
    a = p1s_ref[0]
    k = p2_ref[0]

    sq1 = jnp.sum(a * a, axis=0)
    sq2 = jnp.sum(k * k, axis=0)
    innerm2 = jax.lax.dot_general(
        am2, k, (((0,), (0,)), ((), ())),
        preferred_element_type=jnp.float32,
        precision=jax.lax.Precision.DEFAULT,
    )
    d = (sq1[:, None] + sq2[None, :]) + innerm2

    @pl.when(jnp.logical_and(b == 0, i == 0))
    def _():
        out_ref[0, 0] = 0.0

    d1 = jnp.maximum(jnp.min(d, axis=1), 0.0)
    out_ref[0, 0] += jnp.sum(jnp.sqrt(d1)) * (0.5 / (B * N))

    colmin = jnp.min(d, axis=0)

    @pl.when(i == 0)
    def _():
        d2_ref[0, :] = colmin

    @pl.when(i > 0)
    def _():
        d2_ref[0, :] = jnp.minimum(d2_ref[0, :], colmin)

    @pl.when(i == ni - 1)
    def _():
        d2 = jnp.maximum(d2_ref[0, :], 0.0)
        out_ref[0, 0] += jnp.sum(jnp.sqrt(d2)) * (0.5 / (B * M))


@jax.jit
def kernel(pcs1, pcs2):
    p1t = jnp.transpose(pcs1, (0, 2, 1))
    p2t = jnp.transpose(pcs2, (0, 2, 1))
    p1m2 = p1t * -2.0

    out = pl.pallas_call(
        _chamfer_body,
        grid=(B, N // TN),
        in_specs=[
            pl.BlockSpec((1, 3, TN), lambda b, i: (b, 0, i)),
            pl.BlockSpec((1, 3, TN), lambda b, i: (b, 0, i)),
            pl.BlockSpec((1, 3, M), lambda b, i: (b, 0, 0)),
        ],
        out_specs=pl.BlockSpec(
            (1, 1), lambda b, i: (0, 0), memory_space=pltpu.SMEM
        ),
        out_shape=jax.ShapeDtypeStruct((1, 1), jnp.float32),
        scratch_shapes=[pltpu.VMEM((1, M), jnp.float32)],
    )(p1m2, p1t, p2t)
    return out[0, 0]


# speedup vs baseline: 1.6238x; 1.1039x over previous
import functools

import jax
import jax.numpy as jnp
from jax.experimental import pallas as pl
from jax.experimental.pallas import tpu as pltpu

B = 4
N = 4096
M = 4096
TN = 2048


def _chamfer_body(p1_ref, p1s_ref, p2_ref, out_ref, d2_ref):
    b = pl.program_id(0)
    i = pl.program_id(1)
    ni = pl.num_programs(1)

    am2 = p1_ref[0]
    a = p1s_ref[0]
    k = p2_ref[0]

    sq1 = jnp.sum(a * a, axis=0)
    sq2 = jnp.sum(k * k, axis=0)
    innerm2 = jax.lax.dot_general(
        am2.astype(jnp.bfloat16), k.astype(jnp.bfloat16),
        (((0,), (0,)), ((), ())),
        preferred_element_type=jnp.float32,
        precision=jax.lax.Precision.DEFAULT,
    )
    d = (sq1[:, None] + sq2[None, :]) + innerm2

    @pl.when(jnp.logical_and(b == 0, i == 0))
    def _():
        out_ref[0, 0] = 0.0

    d1 = jnp.maximum(jnp.min(d, axis=1), 0.0)
    out_ref[0, 0] += jnp.sum(jnp.sqrt(d1)) * (0.5 / (B * N))

    colmin = jnp.min(d, axis=0)

    @pl.when(i == 0)
    def _():
        d2_ref[0, :] = colmin

    @pl.when(i > 0)
    def _():
        d2_ref[0, :] = jnp.minimum(d2_ref[0, :], colmin)

    @pl.when(i == ni - 1)
    def _():
        d2 = jnp.maximum(d2_ref[0, :], 0.0)
        out_ref[0, 0] += jnp.sum(jnp.sqrt(d2)) * (0.5 / (B * M))


@jax.jit
def kernel(pcs1, pcs2):
    p1t = jnp.transpose(pcs1, (0, 2, 1))
    p2t = jnp.transpose(pcs2, (0, 2, 1))
    p1m2 = p1t * -2.0

    out = pl.pallas_call(
        _chamfer_body,
        grid=(B, N // TN),
        in_specs=[
            pl.BlockSpec((1, 3, TN), lambda b, i: (b, 0, i)),
            pl.BlockSpec((1, 3, TN), lambda b, i: (b, 0, i)),
            pl.BlockSpec((1, 3, M), lambda b, i: (b, 0, 0)),
        ],
        out_specs=pl.BlockSpec(
            (1, 1), lambda b, i: (0, 0), memory_space=pltpu.SMEM
        ),
        out_shape=jax.ShapeDtypeStruct((1, 1), jnp.float32),
        scratch_shapes=[pltpu.VMEM((1, M), jnp.float32)],
    )(p1m2, p1t, p2t)
    return out[0, 0]
